# baseline (speedup 1.0000x reference)
import functools

import jax
import jax.numpy as jnp
from jax import lax
from jax.experimental import pallas as pl
from jax.experimental.pallas import tpu as pltpu
from jax.experimental.pallas import tpu_sc as plsc


def _make_sc_pool(batch, seq, hid, n_words):
    mesh = plsc.VectorSubcoreMesh(core_axis_name="c", subcore_axis_name="s")
    nw = mesh.num_cores * mesh.num_subcores
    assert batch % nw == 0
    bpw = batch // nw
    assert bpw % 8 == 0 and hid % 16 == 0
    n_vec = hid // 16
    g_rows = 4
    assert seq % (2 * g_rows) == 0
    n_chunks = seq // g_rows

    @functools.partial(
        pl.kernel,
        mesh=mesh,
        out_type=jax.ShapeDtypeStruct((batch, hid), jnp.float32),
        scratch_types=[
            pltpu.VMEM((seq, bpw), jnp.int32),
            pltpu.VMEM((2, g_rows, bpw, hid), jnp.float32),
            pltpu.VMEM((bpw, hid), jnp.float32),
            pltpu.SemaphoreType.DMA,
            pltpu.SemaphoreType.DMA,
        ],
        compiler_params=pltpu.CompilerParams(use_tc_tiling_on_sc=False),
    )
    def sc_pool(x_hbm, emb_hbm, out_hbm, idx_v, rows_v, acc_v, sem0, sem1):
        ncores = mesh.num_cores
        wid = lax.axis_index("s") * ncores + lax.axis_index("c")
        base = wid * bpw

        pltpu.sync_copy(x_hbm.at[:, pl.ds(base, bpw)], idx_v)

        zero = jnp.zeros((16,), jnp.float32)

        @pl.loop(0, bpw)
        def _(b):
            for k in range(n_vec):
                acc_v[b, pl.ds(k * 16, 16)] = zero

        sems = (sem0, sem1)

        def fire(c, buf):
            for g in range(g_rows):
                pltpu.make_async_copy(
                    emb_hbm.at[idx_v.at[c * g_rows + g]],
                    rows_v.at[buf, g],
                    sems[buf],
                ).start()

        def drain(buf):
            for g in range(g_rows):
                pltpu.make_async_copy(
                    emb_hbm.at[idx_v.at[0]],
                    rows_v.at[buf, g],
                    sems[buf],
                ).wait()

        def reduce(buf):
            @pl.loop(0, bpw // 8)
            def _(sub):
                j0 = sub * 8
                acc = [
                    [acc_v[j0 + j, pl.ds(k * 16, 16)] for k in range(n_vec)]
                    for j in range(8)
                ]
                for g in range(g_rows):
                    for j in range(8):
                        for k in range(n_vec):
                            acc[j][k] = acc[j][k] + rows_v[
                                buf, g, j0 + j, pl.ds(k * 16, 16)
                            ]
                for j in range(8):
                    for k in range(n_vec):
                        acc_v[j0 + j, pl.ds(k * 16, 16)] = acc[j][k]

        fire(0, 0)

        @pl.loop(0, n_chunks, step=2)
        def _(c):
            fire(c + 1, 1)
            drain(0)
            reduce(0)

            @pl.when(c + 2 < n_chunks)
            def _():
                fire(c + 2, 0)

            drain(1)
            reduce(1)

        pltpu.sync_copy(acc_v, out_hbm.at[pl.ds(base, bpw)])

    return sc_pool


def _mm_body(inv_seq, h_ref, w_ref, b_ref, o_ref):
    h = h_ref[...] * inv_seq
    o_ref[...] = (
        jnp.dot(h, w_ref[...], preferred_element_type=jnp.float32) + b_ref[...]
    )


def _matmul(hsum, w, b2, inv_seq):
    batch, hid = hsum.shape
    n_class = w.shape[1]
    bm = 512
    grid = (batch // bm,)
    return pl.pallas_call(
        functools.partial(_mm_body, inv_seq),
        grid=grid,
        in_specs=[
            pl.BlockSpec((bm, hid), lambda i: (i, 0)),
            pl.BlockSpec((hid, n_class), lambda i: (0, 0)),
            pl.BlockSpec((1, n_class), lambda i: (0, 0)),
        ],
        out_specs=pl.BlockSpec((bm, n_class), lambda i: (i, 0)),
        out_shape=jax.ShapeDtypeStruct((batch, n_class), jnp.float32),
    )(hsum, w, b2)


def kernel(x, emb, W, b):
    seq, batch = x.shape
    n_words, hid = emb.shape
    sc_pool = _make_sc_pool(batch, seq, hid, n_words)
    hsum = sc_pool(x.astype(jnp.int32), emb)
    y = _matmul(hsum, W, b.reshape(1, -1), 1.0 / seq)
    return y
